# baseline (speedup 1.0000x reference)
import functools

import jax
import jax.numpy as jnp
from jax import lax
from jax.experimental import pallas as pl
from jax.experimental.pallas import tpu as pltpu

_K = 100000
_D = 768
_S = 2048
_KBLK = 800
_SBLK = 256


def _scan_body(q_ref, keys_ref, eps_ref, d2_out, idx_out, eps_out,
               bestd_ref, besti_ref, beste_ref):
    step = pl.program_id(0)

    @pl.when(step == 0)
    def _init():
        bestd_ref[0] = jnp.float32(jnp.inf)
        besti_ref[0] = jnp.int32(0)
        beste_ref[0] = jnp.float32(-1.0)

    q = q_ref[...]
    k = keys_ref[...]
    diff = k - q
    d2 = jnp.sum(diff * diff, axis=1, keepdims=True)
    minv = jnp.min(d2)
    iota = lax.broadcasted_iota(jnp.int32, (_KBLK, 1), 0)
    lidx = jnp.min(jnp.where(d2 == minv, iota, jnp.int32(2**30)))

    eps_blk = eps_ref[...]
    lane = lax.broadcasted_iota(jnp.int32, (1, 1, _KBLK), 2)
    eps_at = jnp.sum(jnp.where(lane == lidx, eps_blk, jnp.float32(0.0)))

    better = minv < bestd_ref[0]
    bestd_ref[0] = jnp.where(better, minv, bestd_ref[0])
    besti_ref[0] = jnp.where(better, step * _KBLK + lidx, besti_ref[0])
    beste_ref[0] = jnp.where(better, eps_at, beste_ref[0])

    @pl.when(step == pl.num_programs(0) - 1)
    def _fin():
        d2_out[0, 0] = bestd_ref[0]
        idx_out[0, 0] = besti_ref[0]
        eps_out[0, 0] = beste_ref[0]


def _nearest_key(query_2d, keys, epsilons):
    eps3d = epsilons.reshape(_K // _KBLK, 1, _KBLK)
    grid = (_K // _KBLK,)
    return pl.pallas_call(
        _scan_body,
        grid=grid,
        in_specs=[
            pl.BlockSpec((1, _D), lambda i: (0, 0)),
            pl.BlockSpec((_KBLK, _D), lambda i: (i, 0)),
            pl.BlockSpec((1, 1, _KBLK), lambda i: (i, 0, 0)),
        ],
        out_specs=[
            pl.BlockSpec(memory_space=pltpu.SMEM),
            pl.BlockSpec(memory_space=pltpu.SMEM),
            pl.BlockSpec(memory_space=pltpu.SMEM),
        ],
        out_shape=[
            jax.ShapeDtypeStruct((1, 1), jnp.float32),
            jax.ShapeDtypeStruct((1, 1), jnp.int32),
            jax.ShapeDtypeStruct((1, 1), jnp.float32),
        ],
        scratch_shapes=[
            pltpu.SMEM((1,), jnp.float32),
            pltpu.SMEM((1,), jnp.int32),
            pltpu.SMEM((1,), jnp.float32),
        ],
    )(query_2d, keys, eps3d)


def _fused_body(d2_ref, idx_ref, eps_ref, tok_ref, x_ref, w_ref, b_ref,
                values_ref, out_ref, val_ref, sem):
    i = pl.program_id(0)

    @pl.when(i == 0)
    def _fetch():
        copy = pltpu.make_async_copy(
            values_ref.at[pl.ds(idx_ref[0, 0], 1)], val_ref, sem)
        copy.start()
        copy.wait()

    out = lax.dot_general(
        x_ref[...], w_ref[...], (((1,), (1,)), ((), ())),
        preferred_element_type=jnp.float32)
    out = out + b_ref[...]

    eps = eps_ref[0, 0]
    cond = (eps >= 0.0) & (d2_ref[0, 0] <= eps * eps)
    rows = i * _SBLK + lax.broadcasted_iota(jnp.int32, (_SBLK, 1), 0)
    mask = (rows < tok_ref[0, 0]) & cond
    out_ref[...] = jnp.where(mask, val_ref[...], out)


def _fused_out(x2d, W, b2d, values, d2, idx, eps, tok):
    grid = (_S // _SBLK,)
    return pl.pallas_call(
        _fused_body,
        grid=grid,
        in_specs=[
            pl.BlockSpec(memory_space=pltpu.SMEM),
            pl.BlockSpec(memory_space=pltpu.SMEM),
            pl.BlockSpec(memory_space=pltpu.SMEM),
            pl.BlockSpec(memory_space=pltpu.SMEM),
            pl.BlockSpec((_SBLK, _D), lambda i: (i, 0)),
            pl.BlockSpec((_D, _D), lambda i: (0, 0)),
            pl.BlockSpec((1, _D), lambda i: (0, 0)),
            pl.BlockSpec(memory_space=pl.ANY),
        ],
        out_specs=pl.BlockSpec((_SBLK, _D), lambda i: (i, 0)),
        out_shape=jax.ShapeDtypeStruct((_S, _D), jnp.float32),
        scratch_shapes=[
            pltpu.VMEM((1, _D), jnp.float32),
            pltpu.SemaphoreType.DMA,
        ],
    )(d2, idx, eps, tok, x2d, W, b2d, values)


def kernel(x, W, b, keys, values, epsilons, key_id):
    tok = jnp.minimum(jnp.asarray(key_id, jnp.int32), x.shape[1] - 1)
    x2d = x[0]
    query = lax.dynamic_slice_in_dim(x2d, tok, 1, axis=0)
    d2, idx, eps = _nearest_key(query, keys, epsilons)
    out = _fused_out(x2d, W, b.reshape(1, _D), values,
                     d2, idx, eps, tok.reshape(1, 1))
    return out[None]


# speedup vs baseline: 1.3128x; 1.3128x over previous
import functools

import jax
import jax.numpy as jnp
from jax import lax
from jax.experimental import pallas as pl
from jax.experimental.pallas import tpu as pltpu

_K = 100000
_D = 768
_S = 2048
_KBLK = 2000
_SBLK = 256


def _scan_body(q_ref, keys_ref, eps_ref, d2_out, idx_out, eps_out,
               bestd_ref, besti_ref, beste_ref):
    step = pl.program_id(0)

    @pl.when(step == 0)
    def _init():
        bestd_ref[0] = jnp.float32(jnp.inf)
        besti_ref[0] = jnp.int32(0)
        beste_ref[0] = jnp.float32(-1.0)

    q = q_ref[...]
    k = keys_ref[...]
    kq = lax.dot_general(k, q, (((1,), (1,)), ((), ())),
                         preferred_element_type=jnp.float32)
    kk = lax.dot_general(k * k, jnp.ones((1, _D), jnp.float32),
                         (((1,), (1,)), ((), ())),
                         preferred_element_type=jnp.float32)
    d2 = kk - 2.0 * kq
    minv = jnp.min(d2)
    iota = lax.broadcasted_iota(jnp.int32, (_KBLK, 1), 0)
    lidx = jnp.min(jnp.where(d2 == minv, iota, jnp.int32(2**30)))

    eps_blk = eps_ref[...]
    lane = lax.broadcasted_iota(jnp.int32, (1, 1, _KBLK), 2)
    eps_at = jnp.sum(jnp.where(lane == lidx, eps_blk, jnp.float32(0.0)))

    better = minv < bestd_ref[0]
    bestd_ref[0] = jnp.where(better, minv, bestd_ref[0])
    besti_ref[0] = jnp.where(better, step * _KBLK + lidx, besti_ref[0])
    beste_ref[0] = jnp.where(better, eps_at, beste_ref[0])

    @pl.when(step == pl.num_programs(0) - 1)
    def _fin():
        q2 = jnp.sum(q_ref[...] * q_ref[...])
        d2_out[0, 0] = bestd_ref[0] + q2
        idx_out[0, 0] = besti_ref[0]
        eps_out[0, 0] = beste_ref[0]


def _nearest_key(query_2d, keys, epsilons):
    eps3d = epsilons.reshape(_K // _KBLK, 1, _KBLK)
    grid = (_K // _KBLK,)
    return pl.pallas_call(
        _scan_body,
        grid=grid,
        in_specs=[
            pl.BlockSpec((1, _D), lambda i: (0, 0)),
            pl.BlockSpec((_KBLK, _D), lambda i: (i, 0)),
            pl.BlockSpec((1, 1, _KBLK), lambda i: (i, 0, 0)),
        ],
        out_specs=[
            pl.BlockSpec(memory_space=pltpu.SMEM),
            pl.BlockSpec(memory_space=pltpu.SMEM),
            pl.BlockSpec(memory_space=pltpu.SMEM),
        ],
        out_shape=[
            jax.ShapeDtypeStruct((1, 1), jnp.float32),
            jax.ShapeDtypeStruct((1, 1), jnp.int32),
            jax.ShapeDtypeStruct((1, 1), jnp.float32),
        ],
        scratch_shapes=[
            pltpu.SMEM((1,), jnp.float32),
            pltpu.SMEM((1,), jnp.int32),
            pltpu.SMEM((1,), jnp.float32),
        ],
    )(query_2d, keys, eps3d)


def _fused_body(d2_ref, idx_ref, eps_ref, tok_ref, x_ref, w_ref, b_ref,
                values_ref, out_ref, val_ref, sem):
    i = pl.program_id(0)

    @pl.when(i == 0)
    def _fetch():
        copy = pltpu.make_async_copy(
            values_ref.at[pl.ds(idx_ref[0, 0], 1)], val_ref, sem)
        copy.start()
        copy.wait()

    out = lax.dot_general(
        x_ref[...], w_ref[...], (((1,), (1,)), ((), ())),
        preferred_element_type=jnp.float32)
    out = out + b_ref[...]

    eps = eps_ref[0, 0]
    cond = (eps >= 0.0) & (d2_ref[0, 0] <= eps * eps)
    rows = i * _SBLK + lax.broadcasted_iota(jnp.int32, (_SBLK, 1), 0)
    mask = (rows < tok_ref[0, 0]) & cond
    out_ref[...] = jnp.where(mask, val_ref[...], out)


def _fused_out(x2d, W, b2d, values, d2, idx, eps, tok):
    grid = (_S // _SBLK,)
    return pl.pallas_call(
        _fused_body,
        grid=grid,
        in_specs=[
            pl.BlockSpec(memory_space=pltpu.SMEM),
            pl.BlockSpec(memory_space=pltpu.SMEM),
            pl.BlockSpec(memory_space=pltpu.SMEM),
            pl.BlockSpec(memory_space=pltpu.SMEM),
            pl.BlockSpec((_SBLK, _D), lambda i: (i, 0)),
            pl.BlockSpec((_D, _D), lambda i: (0, 0)),
            pl.BlockSpec((1, _D), lambda i: (0, 0)),
            pl.BlockSpec(memory_space=pl.ANY),
        ],
        out_specs=pl.BlockSpec((_SBLK, _D), lambda i: (i, 0)),
        out_shape=jax.ShapeDtypeStruct((_S, _D), jnp.float32),
        scratch_shapes=[
            pltpu.VMEM((1, _D), jnp.float32),
            pltpu.SemaphoreType.DMA,
        ],
    )(d2, idx, eps, tok, x2d, W, b2d, values)


def kernel(x, W, b, keys, values, epsilons, key_id):
    tok = jnp.minimum(jnp.asarray(key_id, jnp.int32), x.shape[1] - 1)
    x2d = x[0]
    query = lax.dynamic_slice_in_dim(x2d, tok, 1, axis=0)
    d2, idx, eps = _nearest_key(query, keys, epsilons)
    out = _fused_out(x2d, W, b.reshape(1, _D), values,
                     d2, idx, eps, tok.reshape(1, 1))
    return out[None]


# speedup vs baseline: 1.5039x; 1.1456x over previous
import functools

import jax
import jax.numpy as jnp
from jax import lax
from jax.experimental import pallas as pl
from jax.experimental.pallas import tpu as pltpu

_K = 100000
_D = 768
_S = 2048
_KBLK = 4000
_SBLK = 256


def _scan_body(q_ref, keys_ref, eps_ref, d2_out, idx_out, eps_out,
               bestd_ref, besti_ref, beste_ref):
    step = pl.program_id(0)

    @pl.when(step == 0)
    def _init():
        bestd_ref[0] = jnp.float32(jnp.inf)
        besti_ref[0] = jnp.int32(0)
        beste_ref[0] = jnp.float32(-1.0)

    q = q_ref[...]
    k = keys_ref[...]
    diff = k - q
    d2 = jnp.sum(diff * diff, axis=1, keepdims=True)
    minv = jnp.min(d2)
    iota = lax.broadcasted_iota(jnp.int32, (_KBLK, 1), 0)
    lidx = jnp.min(jnp.where(d2 == minv, iota, jnp.int32(2**30)))

    eps_blk = eps_ref[...]
    lane = lax.broadcasted_iota(jnp.int32, (1, 1, _KBLK), 2)
    eps_at = jnp.sum(jnp.where(lane == lidx, eps_blk, jnp.float32(0.0)))

    better = minv < bestd_ref[0]
    bestd_ref[0] = jnp.where(better, minv, bestd_ref[0])
    besti_ref[0] = jnp.where(better, step * _KBLK + lidx, besti_ref[0])
    beste_ref[0] = jnp.where(better, eps_at, beste_ref[0])

    @pl.when(step == pl.num_programs(0) - 1)
    def _fin():
        d2_out[0, 0] = bestd_ref[0]
        idx_out[0, 0] = besti_ref[0]
        eps_out[0, 0] = beste_ref[0]


def _nearest_key(query_2d, keys, epsilons):
    eps3d = epsilons.reshape(_K // _KBLK, 1, _KBLK)
    grid = (_K // _KBLK,)
    return pl.pallas_call(
        _scan_body,
        grid=grid,
        in_specs=[
            pl.BlockSpec((1, _D), lambda i: (0, 0)),
            pl.BlockSpec((_KBLK, _D), lambda i: (i, 0)),
            pl.BlockSpec((1, 1, _KBLK), lambda i: (i, 0, 0)),
        ],
        out_specs=[
            pl.BlockSpec(memory_space=pltpu.SMEM),
            pl.BlockSpec(memory_space=pltpu.SMEM),
            pl.BlockSpec(memory_space=pltpu.SMEM),
        ],
        out_shape=[
            jax.ShapeDtypeStruct((1, 1), jnp.float32),
            jax.ShapeDtypeStruct((1, 1), jnp.int32),
            jax.ShapeDtypeStruct((1, 1), jnp.float32),
        ],
        scratch_shapes=[
            pltpu.SMEM((1,), jnp.float32),
            pltpu.SMEM((1,), jnp.int32),
            pltpu.SMEM((1,), jnp.float32),
        ],
    )(query_2d, keys, eps3d)


def _fused_body(d2_ref, idx_ref, eps_ref, tok_ref, x_ref, w_ref, b_ref,
                values_ref, out_ref, val_ref, sem):
    i = pl.program_id(0)

    @pl.when(i == 0)
    def _fetch():
        copy = pltpu.make_async_copy(
            values_ref.at[pl.ds(idx_ref[0, 0], 1)], val_ref, sem)
        copy.start()
        copy.wait()

    out = lax.dot_general(
        x_ref[...], w_ref[...], (((1,), (1,)), ((), ())),
        preferred_element_type=jnp.float32)
    out = out + b_ref[...]

    eps = eps_ref[0, 0]
    cond = (eps >= 0.0) & (d2_ref[0, 0] <= eps * eps)
    rows = i * _SBLK + lax.broadcasted_iota(jnp.int32, (_SBLK, 1), 0)
    mask = (rows < tok_ref[0, 0]) & cond
    out_ref[...] = jnp.where(mask, val_ref[...], out)


def _fused_out(x2d, W, b2d, values, d2, idx, eps, tok):
    grid = (_S // _SBLK,)
    return pl.pallas_call(
        _fused_body,
        grid=grid,
        in_specs=[
            pl.BlockSpec(memory_space=pltpu.SMEM),
            pl.BlockSpec(memory_space=pltpu.SMEM),
            pl.BlockSpec(memory_space=pltpu.SMEM),
            pl.BlockSpec(memory_space=pltpu.SMEM),
            pl.BlockSpec((_SBLK, _D), lambda i: (i, 0)),
            pl.BlockSpec((_D, _D), lambda i: (0, 0)),
            pl.BlockSpec((1, _D), lambda i: (0, 0)),
            pl.BlockSpec(memory_space=pl.ANY),
        ],
        out_specs=pl.BlockSpec((_SBLK, _D), lambda i: (i, 0)),
        out_shape=jax.ShapeDtypeStruct((_S, _D), jnp.float32),
        scratch_shapes=[
            pltpu.VMEM((1, _D), jnp.float32),
            pltpu.SemaphoreType.DMA,
        ],
    )(d2, idx, eps, tok, x2d, W, b2d, values)


def kernel(x, W, b, keys, values, epsilons, key_id):
    tok = jnp.minimum(jnp.asarray(key_id, jnp.int32), x.shape[1] - 1)
    x2d = x[0]
    query = lax.dynamic_slice_in_dim(x2d, tok, 1, axis=0)
    d2, idx, eps = _nearest_key(query, keys, epsilons)
    out = _fused_out(x2d, W, b.reshape(1, _D), values,
                     d2, idx, eps, tok.reshape(1, 1))
    return out[None]
